# baseline (speedup 1.0000x reference)
import jax
import jax.numpy as jnp
from jax import lax
from jax.experimental import pallas as pl
from jax.experimental.pallas import tpu as pltpu
from jax.experimental.pallas import tpu_sc as plsc

N = 10000
E = 320000
D = 128
DH = 64
STEPS = 5
NCORE = 2
NTILE = 16
NP = 10240
RPT = NP // NTILE
RBLK = 64
NRB = RPT // RBLK
EPT = E // NTILE
CK = 128
BCH = 8
NCH = 160
NBLK = NCH // BCH
NCHA = NCH + 2 * BCH
LANES = 16
FB = DH // LANES
ZR = 32
GR = NP // DH
GRH = GR // 2
GRT = RPT // DH


def _sc_body(xh, srcp, dstp, y,
             sidxb, didxb, rowbuf, xv, aggv, zv,
             deg_v, degc_v, tmp_v, g_v, h_sp, agg_sp,
             sem_g0, sem_g1, sem_g2, sem_s0, sem_s1, sem_s2,
             sem_is, sem_id):
    sem_g = (sem_g0, sem_g1, sem_g2)
    sem_s = (sem_s0, sem_s1, sem_s2)
    c = lax.axis_index("c")
    t = lax.axis_index("s")
    rows_t = t * RPT
    f32 = jnp.float32
    ones16 = jnp.ones((LANES,), f32)
    zeros16 = jnp.zeros((LANES,), f32)

    def body_zero_z(i, carry):
        for f in range(FB):
            zv[i, pl.ds(f * LANES, LANES)] = zeros16
        return carry
    lax.fori_loop(0, ZR, body_zero_z, 0)

    for half in range(2):
        def body_zero_deg(i, carry):
            for f in range(FB):
                deg_v[i, pl.ds(f * LANES, LANES)] = zeros16
            return carry
        lax.fori_loop(0, GRH, body_zero_deg, 0)

        def body_hist(B, carry, half=half):
            pltpu.sync_copy(dstp.at[t, pl.ds(B * BCH, BCH)], didxb.at[0])
            for m in range(BCH):
                for k in range(CK // LANES):
                    idx = didxb[0, m, pl.ds(k * LANES, LANES)]
                    row = lax.shift_right_logical(idx, 6) - half * GRH
                    msk = jnp.logical_and(row >= 0, row < GRH)
                    plsc.addupdate_scatter(
                        deg_v,
                        [jnp.where(msk, row, 0),
                         jnp.bitwise_and(idx, jnp.int32(DH - 1))],
                        ones16, mask=msk)
            return carry
        lax.fori_loop(0, NBLK, body_hist, 0)
        pltpu.sync_copy(deg_v, agg_sp.at[pl.ds(t * GR + half * GRH, GRH)])
    plsc.subcore_barrier()

    pltpu.sync_copy(agg_sp.at[pl.ds(t * GRT, GRT)], degc_v)
    for u in range(1, NTILE):
        pltpu.sync_copy(agg_sp.at[pl.ds(u * GR + t * GRT, GRT)], tmp_v)

        def body_acc(k, carry):
            for f in range(FB):
                sl = pl.ds(f * LANES, LANES)
                degc_v[k, sl] = degc_v[k, sl] + tmp_v[k, sl]
            return carry
        lax.fori_loop(0, GRT, body_acc, 0)

    def body_g(k, carry):
        for f in range(FB):
            sl = pl.ds(f * LANES, LANES)
            d = degc_v[k, sl]
            i = plsc.bitcast(d, jnp.int32)
            i = jnp.int32(0x5F3759DF) - lax.shift_right_logical(i, 1)
            yb = plsc.bitcast(i, f32)
            hd = 0.5 * d
            for _ in range(4):
                yb = yb * (1.5 - hd * yb * yb)
            g_v[k, sl] = jnp.where(d > 0.5, yb, jnp.zeros_like(yb))
        return carry
    lax.fori_loop(0, GRT, body_g, 0)
    plsc.subcore_barrier()

    def _bg(r_local):
        hi = jnp.full((LANES,), lax.shift_right_logical(r_local, 6), jnp.int32)
        lo = jnp.full((LANES,), jnp.bitwise_and(r_local, DH - 1), jnp.int32)
        return plsc.load_gather(g_v, [hi, lo])

    def _zero_agg_rows(rows0):
        for q in range(RBLK // ZR):
            pltpu.sync_copy(zv, agg_sp.at[pl.ds(rows0 + q * ZR, ZR)])

    for rb in range(NRB):
        rows0 = rows_t + rb * RBLK
        pltpu.sync_copy(xh.at[c, pl.ds(rows0, RBLK)], xv)

        def body_hi(r, carry, rb=rb):
            bg = _bg(rb * RBLK + r)
            for f in range(FB):
                sl = pl.ds(f * LANES, LANES)
                aggv[r, sl] = bg * xv[r, sl]
            return carry
        lax.fori_loop(0, RBLK, body_hi, 0)
        pltpu.sync_copy(aggv, h_sp.at[pl.ds(rows0, RBLK)])
        _zero_agg_rows(rows0)

    def _fire_gather(s, m, b, sem):
        return pltpu.async_copy(h_sp.at[sidxb.at[s, m]], rowbuf.at[b], sem)

    def _wait_gather(b, sem):
        pltpu.make_async_copy(xh.at[0, pl.ds(0, CK)], rowbuf.at[b], sem).wait()

    def _fire_idx(B, s):
        pltpu.async_copy(srcp.at[t, pl.ds(B * BCH, BCH)], sidxb.at[s], sem_is)
        pltpu.async_copy(dstp.at[t, pl.ds(B * BCH, BCH)], didxb.at[s], sem_id)

    def _wait_idx(s):
        pltpu.make_async_copy(srcp.at[t, pl.ds(0, BCH)], sidxb.at[s],
                              sem_is).wait()
        pltpu.make_async_copy(dstp.at[t, pl.ds(0, BCH)], didxb.at[s],
                              sem_id).wait()

    def _fire_scatter(s, m, b, sem):
        return pltpu.async_copy(
            rowbuf.at[b], agg_sp.at[didxb.at[s, m]], sem, add=True)

    def _wait_scatter(b, sem):
        pltpu.make_async_copy(xh.at[0, pl.ds(0, CK)], rowbuf.at[b], sem).wait()

    def _process_block(s):
        _fire_gather(s, 0, 0, sem_g[0])
        _fire_gather(s, 1, 1, sem_g[1])
        for m in range(BCH):
            b = m % 3
            _wait_gather(b, sem_g[b])
            _fire_scatter(s, m, b, sem_s[b])
            if m + 2 < BCH:
                bn = (m + 2) % 3
                if m >= 1:
                    _wait_scatter(bn, sem_s[bn])
                _fire_gather(s, m + 2, bn, sem_g[bn])
        for m in range(BCH - 3, BCH):
            b = m % 3
            _wait_scatter(b, sem_s[b])

    def _scatter_phase():
        _fire_idx(0, 0)
        _wait_idx(0)
        _fire_idx(1, 1)

        def grp(q, carry):
            _process_block(0)
            _fire_idx(2 * q + 2, 0)
            _wait_idx(1)
            _process_block(1)
            _fire_idx(2 * q + 3, 1)
            _wait_idx(0)
            return carry
        lax.fori_loop(0, NBLK // 2, grp, 0)
        _wait_idx(1)

    def _phase_e(final):
        for rb in range(NRB):
            rows0 = rows_t + rb * RBLK
            pltpu.async_copy(agg_sp.at[pl.ds(rows0, RBLK)], aggv, sem_g0)
            pltpu.async_copy(xh.at[c, pl.ds(rows0, RBLK)], xv, sem_g1)
            pltpu.make_async_copy(
                xh.at[0, pl.ds(0, RBLK)], aggv, sem_g0).wait()
            pltpu.make_async_copy(
                xh.at[0, pl.ds(0, RBLK)], xv, sem_g1).wait()
            if not final:
                _zero_agg_rows(rows0)

            def body(r, carry, rb=rb):
                bg = _bg(rb * RBLK + r)
                for f in range(FB):
                    sl = pl.ds(f * LANES, LANES)
                    v = bg * aggv[r, sl] + xv[r, sl]
                    if final:
                        aggv[r, sl] = 0.5 * v
                    else:
                        aggv[r, sl] = (0.5 * bg) * v
                return carry
            lax.fori_loop(0, RBLK, body, 0)
            if final:
                pltpu.sync_copy(aggv, y.at[c, pl.ds(rows0, RBLK)])
            else:
                pltpu.sync_copy(aggv, h_sp.at[pl.ds(rows0, RBLK)])

    plsc.subcore_barrier()

    def step_body(k, carry):
        _scatter_phase()
        plsc.subcore_barrier()
        _phase_e(False)
        plsc.subcore_barrier()
        return carry
    lax.fori_loop(0, STEPS - 1, step_body, 0)
    _scatter_phase()
    plsc.subcore_barrier()
    _phase_e(True)


def _sc_call(xh, srcp, dstp):
    mesh = plsc.VectorSubcoreMesh(
        core_axis_name="c", subcore_axis_name="s",
        num_cores=NCORE, num_subcores=NTILE)
    fn = pl.kernel(
        _sc_body,
        out_type=[
            jax.ShapeDtypeStruct((NCORE, NP, DH), jnp.float32),
        ],
        mesh=mesh,
        compiler_params=pltpu.CompilerParams(
            needs_layout_passes=False, use_tc_tiling_on_sc=False),
        scratch_types=[
            pltpu.VMEM((2, BCH, CK), jnp.int32),
            pltpu.VMEM((2, BCH, CK), jnp.int32),
            pltpu.VMEM((3, CK, DH), jnp.float32),
            pltpu.VMEM((RBLK, DH), jnp.float32),
            pltpu.VMEM((RBLK, DH), jnp.float32),
            pltpu.VMEM((ZR, DH), jnp.float32),
            pltpu.VMEM((GRH, DH), jnp.float32),
            pltpu.VMEM((GRT, DH), jnp.float32),
            pltpu.VMEM((GRT, DH), jnp.float32),
            pltpu.VMEM((GRT, DH), jnp.float32),
            pltpu.VMEM_SHARED((NP, DH), jnp.float32),
            pltpu.VMEM_SHARED((NP, DH), jnp.float32),
            pltpu.SemaphoreType.DMA,
            pltpu.SemaphoreType.DMA,
            pltpu.SemaphoreType.DMA,
            pltpu.SemaphoreType.DMA,
            pltpu.SemaphoreType.DMA,
            pltpu.SemaphoreType.DMA,
            pltpu.SemaphoreType.DMA,
            pltpu.SemaphoreType.DMA,
        ],
    )
    return fn(xh, srcp, dstp)


def kernel(x, edge_index):
    src = edge_index[0].astype(jnp.int32)
    dst = edge_index[1].astype(jnp.int32)

    def prep(e):
        e = e.reshape(NTILE, EPT)
        pad = jnp.full((NTILE, NCHA * CK - EPT), NP - 1, jnp.int32)
        return jnp.concatenate([e, pad], axis=1).reshape(NTILE, NCHA, CK)

    srcp = prep(src)
    dstp = prep(dst)
    xh = jnp.zeros((NCORE, NP, DH), jnp.float32)
    xh = xh.at[0, :N, :].set(x[:, :DH]).at[1, :N, :].set(x[:, DH:])
    (yh,) = _sc_call(xh, srcp, dstp)
    return jnp.concatenate([yh[0, :N], yh[1, :N]], axis=1)


# speedup vs baseline: 1.0973x; 1.0973x over previous
import jax
import jax.numpy as jnp
from jax import lax
from jax.experimental import pallas as pl
from jax.experimental.pallas import tpu as pltpu
from jax.experimental.pallas import tpu_sc as plsc

N = 10000
E = 320000
D = 128
DH = 64
STEPS = 5
NCORE = 2
NTILE = 16
NP = 10240
RPT = NP // NTILE
RBLK = 32
NRB = RPT // RBLK
EPT = E // NTILE
CK = 128
BCH = 16
NCH = 160
NBLK = NCH // BCH
NCHA = NCH + 2 * BCH
LANES = 16
FB = DH // LANES
ZR = 32
GR = NP // DH
GRT = RPT // DH


def _sc_body(xh, srcp, dstp, y,
             sidxb, didxb, rowbuf, xv, aggv, zv,
             deg_v, degc_v, tmp_v, g_v, h_sp, agg_sp,
             sem_g0, sem_g1, sem_is, sem_id):
    c = lax.axis_index("c")
    t = lax.axis_index("s")
    rows_t = t * RPT
    f32 = jnp.float32
    ones16 = jnp.ones((LANES,), f32)
    zeros16 = jnp.zeros((LANES,), f32)

    def body_zero_z(i, carry):
        for f in range(FB):
            zv[i, pl.ds(f * LANES, LANES)] = zeros16
        return carry
    lax.fori_loop(0, RBLK, body_zero_z, 0)

    def body_zero_deg(i, carry):
        for f in range(FB):
            deg_v[i, pl.ds(f * LANES, LANES)] = zeros16
        return carry
    lax.fori_loop(0, GR, body_zero_deg, 0)

    def body_hist(B, carry):
        pltpu.sync_copy(dstp.at[t, pl.ds(B * BCH, BCH)], didxb.at[0])
        for m in range(BCH):
            for k in range(CK // LANES):
                idx = didxb[0, m, pl.ds(k * LANES, LANES)]
                plsc.addupdate_scatter(
                    deg_v,
                    [lax.shift_right_logical(idx, 6),
                     jnp.bitwise_and(idx, jnp.int32(DH - 1))],
                    ones16)
        return carry
    lax.fori_loop(0, NBLK, body_hist, 0)

    pltpu.sync_copy(deg_v, agg_sp.at[pl.ds(t * GR, GR)])
    plsc.subcore_barrier()

    pltpu.sync_copy(agg_sp.at[pl.ds(t * GRT, GRT)], degc_v)
    for u in range(1, NTILE):
        pltpu.sync_copy(agg_sp.at[pl.ds(u * GR + t * GRT, GRT)], tmp_v)

        def body_acc(k, carry):
            for f in range(FB):
                sl = pl.ds(f * LANES, LANES)
                degc_v[k, sl] = degc_v[k, sl] + tmp_v[k, sl]
            return carry
        lax.fori_loop(0, GRT, body_acc, 0)

    def body_g(k, carry):
        for f in range(FB):
            sl = pl.ds(f * LANES, LANES)
            d = degc_v[k, sl]
            i = plsc.bitcast(d, jnp.int32)
            i = jnp.int32(0x5F3759DF) - lax.shift_right_logical(i, 1)
            yb = plsc.bitcast(i, f32)
            hd = 0.5 * d
            for _ in range(4):
                yb = yb * (1.5 - hd * yb * yb)
            g_v[k, sl] = jnp.where(d > 0.5, yb, jnp.zeros_like(yb))
        return carry
    lax.fori_loop(0, GRT, body_g, 0)
    plsc.subcore_barrier()

    def _bg(r_local):
        hi = jnp.full((LANES,), lax.shift_right_logical(r_local, 6), jnp.int32)
        lo = jnp.full((LANES,), jnp.bitwise_and(r_local, DH - 1), jnp.int32)
        return plsc.load_gather(g_v, [hi, lo])

    def _zero_agg_rows(rows0):
        pltpu.sync_copy(zv, agg_sp.at[pl.ds(rows0, RBLK)])

    def body_hinit(rb, carry):
        rows0 = rows_t + rb * RBLK
        pltpu.sync_copy(xh.at[c, pl.ds(rows0, RBLK)], xv.at[0])

        def body_hi(r, carry2):
            bg = _bg(rb * RBLK + r)
            for f in range(FB):
                sl = pl.ds(f * LANES, LANES)
                aggv[0, r, sl] = bg * xv[0, r, sl]
            return carry2
        lax.fori_loop(0, RBLK, body_hi, 0)
        pltpu.sync_copy(aggv.at[0], h_sp.at[pl.ds(rows0, RBLK)])
        _zero_agg_rows(rows0)
        return carry
    lax.fori_loop(0, NRB, body_hinit, 0)

    def _fire_gather(s, m, b, sem):
        return pltpu.async_copy(h_sp.at[sidxb.at[s, m]], rowbuf.at[b], sem)

    def _wait_gather(b, sem):
        pltpu.make_async_copy(xh.at[0, pl.ds(0, CK)], rowbuf.at[b], sem).wait()

    def _fire_idx(B, s):
        pltpu.async_copy(srcp.at[t, pl.ds(B * BCH, BCH)], sidxb.at[s], sem_is)
        pltpu.async_copy(dstp.at[t, pl.ds(B * BCH, BCH)], didxb.at[s], sem_id)

    def _wait_idx(s):
        pltpu.make_async_copy(srcp.at[t, pl.ds(0, BCH)], sidxb.at[s],
                              sem_is).wait()
        pltpu.make_async_copy(dstp.at[t, pl.ds(0, BCH)], didxb.at[s],
                              sem_id).wait()

    def _process_block(s):
        _fire_gather(s, 0, 0, sem_g0)
        for m in range(BCH):
            b = m & 1
            sem = sem_g0 if b == 0 else sem_g1
            _wait_gather(b, sem)
            if m + 1 < BCH:
                nsem = sem_g1 if b == 0 else sem_g0
                _fire_gather(s, m + 1, b ^ 1, nsem)
            pltpu.sync_copy(rowbuf.at[b], agg_sp.at[didxb.at[s, m]],
                            add=True)

    def _scatter_phase():
        _fire_idx(0, 0)
        _wait_idx(0)
        _fire_idx(1, 1)

        def grp(q, carry):
            _process_block(0)
            _fire_idx(2 * q + 2, 0)
            _wait_idx(1)
            _process_block(1)
            _fire_idx(2 * q + 3, 1)
            _wait_idx(0)
            return carry
        lax.fori_loop(0, NBLK // 2, grp, 0)
        _wait_idx(1)

    def _load_blk(rb, s):
        rows0 = rows_t + rb * RBLK
        sa, sb = (sem_g0, sem_g1) if s == 0 else (sem_is, sem_id)
        pltpu.async_copy(agg_sp.at[pl.ds(rows0, RBLK)], aggv.at[s], sa)
        pltpu.async_copy(xh.at[c, pl.ds(rows0, RBLK)], xv.at[s], sb)

    def _wait_blk(s):
        sa, sb = (sem_g0, sem_g1) if s == 0 else (sem_is, sem_id)
        pltpu.make_async_copy(xh.at[0, pl.ds(0, RBLK)], aggv.at[s], sa).wait()
        pltpu.make_async_copy(xh.at[0, pl.ds(0, RBLK)], xv.at[s], sb).wait()

    def _phase_e(final):
        _load_blk(0, 0)

        def eblk(q, carry):
            for s in range(2):
                rb = 2 * q + s
                rows0 = rows_t + rb * RBLK
                _wait_blk(s)
                _load_blk(jnp.minimum(rb + 1, NRB - 1), s ^ 1)
                if not final:
                    _zero_agg_rows(rows0)

                def body(r, carry2, s=s, rb=rb):
                    bg = _bg(rb * RBLK + r)
                    for f in range(FB):
                        sl = pl.ds(f * LANES, LANES)
                        v = bg * aggv[s, r, sl] + xv[s, r, sl]
                        if final:
                            aggv[s, r, sl] = 0.5 * v
                        else:
                            aggv[s, r, sl] = (0.5 * bg) * v
                    return carry2
                lax.fori_loop(0, RBLK, body, 0)
                if final:
                    pltpu.sync_copy(aggv.at[s], y.at[c, pl.ds(rows0, RBLK)])
                else:
                    pltpu.sync_copy(aggv.at[s], h_sp.at[pl.ds(rows0, RBLK)])
            return carry
        lax.fori_loop(0, NRB // 2, eblk, 0)
        _wait_blk(0)

    plsc.subcore_barrier()

    def step_body(k, carry):
        _scatter_phase()
        plsc.subcore_barrier()
        _phase_e(False)
        plsc.subcore_barrier()
        return carry
    lax.fori_loop(0, STEPS - 1, step_body, 0)
    _scatter_phase()
    plsc.subcore_barrier()
    _phase_e(True)


def _sc_call(xh, srcp, dstp):
    mesh = plsc.VectorSubcoreMesh(
        core_axis_name="c", subcore_axis_name="s",
        num_cores=NCORE, num_subcores=NTILE)
    fn = pl.kernel(
        _sc_body,
        out_type=[
            jax.ShapeDtypeStruct((NCORE, NP, DH), jnp.float32),
        ],
        mesh=mesh,
        compiler_params=pltpu.CompilerParams(
            needs_layout_passes=False, use_tc_tiling_on_sc=False),
        scratch_types=[
            pltpu.VMEM((2, BCH, CK), jnp.int32),
            pltpu.VMEM((2, BCH, CK), jnp.int32),
            pltpu.VMEM((2, CK, DH), jnp.float32),
            pltpu.VMEM((2, RBLK, DH), jnp.float32),
            pltpu.VMEM((2, RBLK, DH), jnp.float32),
            pltpu.VMEM((ZR, DH), jnp.float32),
            pltpu.VMEM((GR, DH), jnp.float32),
            pltpu.VMEM((GRT, DH), jnp.float32),
            pltpu.VMEM((GRT, DH), jnp.float32),
            pltpu.VMEM((GRT, DH), jnp.float32),
            pltpu.VMEM_SHARED((NP, DH), jnp.float32),
            pltpu.VMEM_SHARED((NP, DH), jnp.float32),
            pltpu.SemaphoreType.DMA,
            pltpu.SemaphoreType.DMA,
            pltpu.SemaphoreType.DMA,
            pltpu.SemaphoreType.DMA,
        ],
    )
    return fn(xh, srcp, dstp)


def kernel(x, edge_index):
    src = edge_index[0].astype(jnp.int32)
    dst = edge_index[1].astype(jnp.int32)

    def prep(e):
        e = e.reshape(NTILE, EPT)
        pad = jnp.full((NTILE, NCHA * CK - EPT), NP - 1, jnp.int32)
        return jnp.concatenate([e, pad], axis=1).reshape(NTILE, NCHA, CK)

    srcp = prep(src)
    dstp = prep(dst)
    xh = jnp.zeros((NCORE, NP, DH), jnp.float32)
    xh = xh.at[0, :N, :].set(x[:, :DH]).at[1, :N, :].set(x[:, DH:])
    (yh,) = _sc_call(xh, srcp, dstp)
    return jnp.concatenate([yh[0, :N], yh[1, :N]], axis=1)
